# Initial kernel scaffold; baseline (speedup 1.0000x reference)
import jax
import jax.numpy as jnp
from jax.experimental import pallas as pl


def kernel(mask, hor_rows, hor_cols, hor_vals, ver_rows, ver_cols, ver_vals, weights1, weights2, bias2, node_idx):
    raise NotImplementedError("write your pallas kernel here")


# speedup vs baseline: 7.2540x; 7.2540x over previous
import functools

import jax
import jax.numpy as jnp
from jax import lax
from jax.experimental import pallas as pl
from jax.experimental.pallas import tpu as pltpu
from jax.experimental.pallas import tpu_sc as plsc

NC = 2
NS = 16
L = 16

_SC_PARAMS = pltpu.CompilerParams(needs_layout_passes=False, use_tc_tiling_on_sc=False)


def _mesh():
  return plsc.VectorSubcoreMesh(core_axis_name="c", subcore_axis_name="s")


def _fused_body(n, npad, sub, nsub, ur, r_rel, emb,
                mask_hbm, hr_hbm, hc_hbm, hv_hbm, vr_hbm, vc_hbm, vv_hbm,
                w1_hbm, node_hbm, z_hbm,
                mh_hbm, mv_hbm, u_hbm, pp_hbm,
                fa_v, fb_v, mo_v, ia_v, ib_v, ca_v, cb_v, nd_v, rows_v,
                idx_v, dst_v, node_v, ones_v,
                nd_sh, hacc_sh, u_sh, sem):
  cid = lax.axis_index("c")
  sid = lax.axis_index("s")
  sl = npad // NS
  vsh = nsub * sub
  rep_base = sid * vsh
  lane = lax.iota(jnp.int32, L)

  node_cp = pltpu.async_copy(node_hbm, node_v, sem)
  ones_v[...] = jnp.ones((L,), jnp.int32)

  @pl.loop(0, sl, step=L)
  def _(i):
    ca_v[pl.ds(i, L)] = jnp.zeros((L,), jnp.int32)

  pltpu.sync_copy(ca_v.at[pl.ds(0, sl)], nd_sh.at[pl.ds(sid * sl, sl)])
  pltpu.sync_copy(z_hbm, hacc_sh.at[pl.ds(sid * sl, sl), :])

  @pl.when(sid == 0)
  def _():
    pltpu.sync_copy(z_hbm.at[pl.ds(0, ur), :], u_sh)

  node_cp.wait()
  node = node_v[...]
  plsc.subcore_barrier()

  cnts = []
  for g in range(nsub):
    b = rep_base + g * sub
    cps = [
        pltpu.async_copy(mask_hbm.at[pl.ds(b, sub)], fa_v, sem),
        pltpu.async_copy(vv_hbm.at[pl.ds(b, sub)], fb_v, sem),
        pltpu.async_copy(vr_hbm.at[pl.ds(b, sub)], ia_v, sem),
        pltpu.async_copy(vc_hbm.at[pl.ds(b, sub)], ib_v, sem),
    ]
    for c in cps:
      c.wait()

    def sbody(i, cnt):
      s = pl.ds(i * L, L)
      sym = 1.0 / (1.0 + jnp.exp(-fa_v[s]))
      mvx = fb_v[s] * sym
      mo_v[s] = mvx
      vr = ia_v[s]
      sel = jnp.logical_and(vr % n == node, mvx > 0.5)
      ap = plsc.all_reduce_population_count(sel)

      @pl.when(ap[0] != 0)
      def _():
        inc = plsc.cumsum(jnp.where(sel, 1, 0).astype(jnp.int32))
        pos = cnt + inc - 1
        plsc.store_scatter(ca_v, [pos], (vr // n) * n + ib_v[s], mask=sel)

      return cnt + ap

    cnt = lax.fori_loop(0, sub // L, sbody, jnp.zeros((L,), jnp.int32))
    cnts.append(cnt)

    @pl.when((sid * nsub + g) // NS == cid)
    def _():
      pltpu.sync_copy(mo_v, mv_hbm.at[pl.ds(b, sub)])

    def bbody(k, carry):
      valid = (k * L + lane) < cnt
      idx_v[...] = jnp.where(valid, ca_v[pl.ds(k * L, L)] % n, n)
      pltpu.sync_copy(ones_v, nd_sh.at[idx_v])
      return carry

    lax.fori_loop(0, (cnt[0] + (L - 1)) // L, bbody, 0)

    slot = (sid * nsub + g) * sub
    pltpu.sync_copy(ca_v, pp_hbm.at[pl.ds(slot, sub)])

  plsc.subcore_barrier()

  pltpu.sync_copy(nd_sh, nd_v)
  for g in range(nsub):
    b = rep_base + g * sub
    cps = [
        pltpu.async_copy(mask_hbm.at[pl.ds(b, sub)], fa_v, sem),
        pltpu.async_copy(hv_hbm.at[pl.ds(b, sub)], fb_v, sem),
        pltpu.async_copy(hr_hbm.at[pl.ds(b, sub)], ia_v, sem),
        pltpu.async_copy(hc_hbm.at[pl.ds(b, sub)], ib_v, sem),
    ]
    for c in cps:
      c.wait()

    def hbody(i, cnt):
      s = pl.ds(i * L, L)
      sym = 1.0 / (1.0 + jnp.exp(-fa_v[s]))
      mhx = fb_v[s] * sym
      mo_v[s] = mhx
      hr = ia_v[s]
      ndl = plsc.load_gather(nd_v, [hr])
      active = jnp.logical_and(mhx > 0.5, ndl > 0)
      ap = plsc.all_reduce_population_count(active)

      @pl.when(ap[0] != 0)
      def _():
        inc = plsc.cumsum(jnp.where(active, 1, 0).astype(jnp.int32))
        pos = cnt + inc - 1
        plsc.store_scatter(ca_v, [pos], ib_v[s], mask=active)
        plsc.store_scatter(cb_v, [pos], hr, mask=active)

      return cnt + ap

    hcnt = lax.fori_loop(0, sub // L, hbody, jnp.zeros((L,), jnp.int32))

    @pl.when((sid * nsub + g) // NS == cid)
    def _():
      pltpu.sync_copy(mo_v, mh_hbm.at[pl.ds(b, sub)])

    def gbody(k, carry):
      s = pl.ds(k * L, L)
      valid = (k * L + lane) < hcnt
      idx_v[...] = jnp.where(valid, ca_v[s], 0)
      dst_v[...] = jnp.where(valid, cb_v[s], n)
      pltpu.sync_copy(w1_hbm.at[idx_v], rows_v)
      pltpu.sync_copy(rows_v, hacc_sh.at[dst_v], add=True)
      return carry

    lax.fori_loop(0, (hcnt[0] + (L - 1)) // L, gbody, 0)

  plsc.subcore_barrier()

  for g in range(nsub):
    cnt = cnts[g]
    slot = (sid * nsub + g) * sub

    def cbody(k, carry):
      pltpu.sync_copy(pp_hbm.at[pl.ds(slot + k * L, L)], idx_v)
      valid = (k * L + lane) < cnt
      p = idx_v[...]
      idx_v[...] = jnp.where(valid, p % n, 0)
      dst_v[...] = jnp.where(valid, p // n, r_rel)
      pltpu.sync_copy(hacc_sh.at[idx_v], rows_v)

      @pl.loop(0, L)
      def _(i):
        @pl.loop(0, emb, step=L)
        def _(j):
          rows_v[i, pl.ds(j, L)] = jnp.maximum(rows_v[i, pl.ds(j, L)], 0.0)

      pltpu.sync_copy(rows_v, u_sh.at[dst_v], add=True)
      return carry

    lax.fori_loop(0, (cnt[0] + (L - 1)) // L, cbody, 0)

  plsc.subcore_barrier()

  @pl.when(jnp.logical_and(cid == 0, sid == 0))
  def _():
    pltpu.sync_copy(u_sh, u_hbm)


def _k4_body(r_rel, u_ref, w2_ref, b_ref, o_ref):
  u = u_ref[...][:r_rel]
  prod = u[:, :, None] * w2_ref[...]
  s1 = jnp.sum(prod, axis=0)
  logits = jnp.sum(s1, axis=0, keepdims=True) + b_ref[...]
  m = jnp.max(logits, axis=1, keepdims=True)
  e = jnp.exp(logits - m)
  o_ref[...] = e / jnp.sum(e, axis=1, keepdims=True)


def kernel(mask, hor_rows, hor_cols, hor_vals, ver_rows, ver_cols, ver_vals,
           weights1, weights2, bias2, node_idx):
  e = mask.shape[0]
  rn, emb = weights1.shape
  r_rel, _, c_cls = weights2.shape
  n = rn // r_rel
  sub = e // (NC * NS)
  nsub = NC
  assert sub * NS * nsub == e and sub % L == 0
  npad = (n // (NS * L) + 1) * (NS * L)
  ur = ((r_rel + L) // L) * L

  f32, i32 = jnp.float32, jnp.int32
  node_vec = jnp.full((L,), node_idx, i32)
  zrows = jnp.zeros((npad // NS, emb), f32)
  hr = hor_rows.astype(i32)
  hc = hor_cols.astype(i32)
  vr = ver_rows.astype(i32)
  vc = ver_cols.astype(i32)

  fused = pl.kernel(
      functools.partial(_fused_body, n, npad, sub, nsub, ur, r_rel, emb),
      out_type=(jax.ShapeDtypeStruct((e,), f32),
                jax.ShapeDtypeStruct((e,), f32),
                jax.ShapeDtypeStruct((ur, emb), f32),
                jax.ShapeDtypeStruct((e,), i32)),
      mesh=_mesh(),
      scratch_types=[
          pltpu.VMEM((sub,), f32), pltpu.VMEM((sub,), f32),
          pltpu.VMEM((sub,), f32),
          pltpu.VMEM((sub,), i32), pltpu.VMEM((sub,), i32),
          pltpu.VMEM((sub,), i32), pltpu.VMEM((sub,), i32),
          pltpu.VMEM((npad,), i32),
          pltpu.VMEM((L, emb), f32),
          pltpu.VMEM((L,), i32), pltpu.VMEM((L,), i32),
          pltpu.VMEM((L,), i32), pltpu.VMEM((L,), i32),
          pltpu.VMEM_SHARED((npad,), i32),
          pltpu.VMEM_SHARED((npad, emb), f32),
          pltpu.VMEM_SHARED((ur, emb), f32),
          pltpu.SemaphoreType.DMA,
      ],
      compiler_params=_SC_PARAMS)
  mh, mv, u, _pp = fused(mask, hr, hc, hor_vals, vr, vc, ver_vals,
                         weights1, node_vec, zrows)

  res2d = pl.pallas_call(
      functools.partial(_k4_body, r_rel),
      out_shape=jax.ShapeDtypeStruct((1, c_cls), f32),
  )(u, weights2, bias2.reshape(1, c_cls))

  return (res2d.reshape(c_cls), mh, mv)
